# baseline (device time: 23309 ns/iter reference)
import jax
import jax.numpy as jnp
from jax import lax
from jax.experimental import pallas as pl
from jax.experimental.pallas import tpu as pltpu

C = 8


def kernel(x):
    _, m, n = x.shape
    half_n = n // 2
    half_m = m // 2
    rows = half_m // C

    def body(x_ref, out_ref, ybuf, xbuf, own_buf, send_buf, res_buf,
             y_send_sems, y_recv_sems, x_send_sems, x_recv_sems,
             in_sems, out_sems):
        my_x = lax.axis_index("x")
        my_y = lax.axis_index("y")
        my_z = lax.axis_index("z")
        y_partner = (my_x, 1 - my_y, my_z)
        x_partner = (1 - my_x, my_y, my_z)

        my_cols = pl.ds(my_y * half_n, half_n)
        partner_cols = pl.ds((1 - my_y) * half_n, half_n)

        stage_send = pltpu.make_async_copy(
            x_ref.at[0, pl.ds(my_x * half_m, half_m), partner_cols],
            send_buf, in_sems.at[0],
        )
        stage_own = pltpu.make_async_copy(
            x_ref.at[0, :, my_cols], own_buf, in_sems.at[1],
        )
        stage_send.start()
        stage_own.start()

        barrier_sem = pltpu.get_barrier_semaphore()
        for nbr in (y_partner, x_partner):
            pl.semaphore_signal(
                barrier_sem, inc=1,
                device_id=nbr, device_id_type=pl.DeviceIdType.MESH,
            )
        pl.semaphore_wait(barrier_sem, 2)
        stage_send.wait()

        y_rdmas = []
        for c in range(C):
            rdma = pltpu.make_async_remote_copy(
                src_ref=send_buf.at[pl.ds(c * rows, rows), :],
                dst_ref=ybuf.at[pl.ds(c * rows, rows), :],
                send_sem=y_send_sems.at[c],
                recv_sem=y_recv_sems.at[c],
                device_id=y_partner,
                device_id_type=pl.DeviceIdType.MESH,
            )
            rdma.start()
            y_rdmas.append(rdma)

        stage_own.wait()

        x_rdmas = []
        out_copies = []
        for c in range(C):
            y_rdmas[c].wait_recv()
            rdma = pltpu.make_async_remote_copy(
                src_ref=ybuf.at[pl.ds(c * rows, rows), :],
                dst_ref=xbuf.at[pl.ds(c * rows, rows), :],
                send_sem=x_send_sems.at[c],
                recv_sem=x_recv_sems.at[c],
                device_id=x_partner,
                device_id_type=pl.DeviceIdType.MESH,
            )
            rdma.start()
            x_rdmas.append(rdma)
            my_rows = pl.ds(my_x * half_m + c * rows, rows)
            res_buf[my_rows, :] = (
                own_buf[my_rows, :] + ybuf[pl.ds(c * rows, rows), :]
            )
            wb = pltpu.make_async_copy(
                res_buf.at[my_rows, :], out_ref.at[my_rows, :], out_sems.at[c]
            )
            wb.start()
            out_copies.append(wb)

        for c in range(C):
            x_rdmas[c].wait_recv()
            nbr_rows = pl.ds((1 - my_x) * half_m + c * rows, rows)
            res_buf[nbr_rows, :] = (
                own_buf[nbr_rows, :] + xbuf[pl.ds(c * rows, rows), :]
            )
            wb = pltpu.make_async_copy(
                res_buf.at[nbr_rows, :], out_ref.at[nbr_rows, :],
                out_sems.at[C + c],
            )
            wb.start()
            out_copies.append(wb)

        for wb in out_copies:
            wb.wait()
        for c in range(C):
            y_rdmas[c].wait_send()
            x_rdmas[c].wait_send()

    return pl.pallas_call(
        body,
        out_shape=jax.ShapeDtypeStruct((m, half_n), jnp.float32),
        in_specs=[pl.BlockSpec(memory_space=pl.ANY)],
        out_specs=pl.BlockSpec(memory_space=pl.ANY),
        scratch_shapes=[
            pltpu.VMEM((half_m, half_n), jnp.float32),
            pltpu.VMEM((half_m, half_n), jnp.float32),
            pltpu.VMEM((m, half_n), jnp.float32),
            pltpu.VMEM((half_m, half_n), jnp.float32),
            pltpu.VMEM((m, half_n), jnp.float32),
            pltpu.SemaphoreType.DMA((C,)),
            pltpu.SemaphoreType.DMA((C,)),
            pltpu.SemaphoreType.DMA((C,)),
            pltpu.SemaphoreType.DMA((C,)),
            pltpu.SemaphoreType.DMA((2,)),
            pltpu.SemaphoreType.DMA((2 * C,)),
        ],
        compiler_params=pltpu.CompilerParams(collective_id=0),
    )(x)


# device time: 21425 ns/iter; 1.0879x vs baseline; 1.0879x over previous
import jax
import jax.numpy as jnp
from jax import lax
from jax.experimental import pallas as pl
from jax.experimental.pallas import tpu as pltpu

C = 8
H = C // 2


def kernel(x):
    _, m, n = x.shape
    half_n = n // 2
    quarter = m // 4
    rows = quarter // C

    def body(x_ref, out_ref, ybuf, xbuf, zbuf, dbuf,
             ys, yr, xs, xr, zs, zr, fxs, fzs, dxr, dzr):
        my_x = lax.axis_index("x")
        my_y = lax.axis_index("y")
        my_z = lax.axis_index("z")
        p = my_z % 2
        y_partner = (my_x, 1 - my_y, my_z)
        x_partner = (1 - my_x, my_y, my_z)
        z_partner = (my_x, my_y, my_z + 1 - 2 * p)

        q_me = 2 * my_x + p
        q_xn = 2 * (1 - my_x) + p
        q_zn = 2 * my_x + (1 - p)
        q_dg = 2 * (1 - my_x) + (1 - p)

        my_cols = pl.ds(my_y * half_n, half_n)
        partner_cols = pl.ds((1 - my_y) * half_n, half_n)

        def qrows(q, c):
            return pl.ds(q * quarter + c * rows, rows)

        def crows(c):
            return pl.ds(c * rows, rows)

        barrier_sem = pltpu.get_barrier_semaphore()
        for nbr in (y_partner, x_partner, z_partner):
            pl.semaphore_signal(
                barrier_sem, inc=1,
                device_id=nbr, device_id_type=pl.DeviceIdType.MESH,
            )
        pl.semaphore_wait(barrier_sem, 3)

        y_rdmas = []
        for c in range(C):
            rdma = pltpu.make_async_remote_copy(
                src_ref=x_ref.at[0, qrows(q_me, c), partner_cols],
                dst_ref=ybuf.at[crows(c)],
                send_sem=ys.at[c], recv_sem=yr.at[c],
                device_id=y_partner, device_id_type=pl.DeviceIdType.MESH,
            )
            rdma.start()
            y_rdmas.append(rdma)

        swap_x, swap_z = [], []
        for c in range(C):
            y_rdmas[c].wait_recv()
            sx = pltpu.make_async_remote_copy(
                src_ref=ybuf.at[crows(c)], dst_ref=xbuf.at[crows(c)],
                send_sem=xs.at[c], recv_sem=xr.at[c],
                device_id=x_partner, device_id_type=pl.DeviceIdType.MESH,
            )
            sx.start()
            swap_x.append(sx)
            sz = pltpu.make_async_remote_copy(
                src_ref=ybuf.at[crows(c)], dst_ref=zbuf.at[crows(c)],
                send_sem=zs.at[c], recv_sem=zr.at[c],
                device_id=z_partner, device_id_type=pl.DeviceIdType.MESH,
            )
            sz.start()
            swap_z.append(sz)
            out_ref[qrows(q_me, c), :] = (
                x_ref[0, qrows(q_me, c), my_cols] + ybuf[crows(c), :]
            )

        fwd_x = []
        for c in range(C):
            swap_z[c].wait_recv()
            if c < H:
                fx = pltpu.make_async_remote_copy(
                    src_ref=zbuf.at[crows(c)], dst_ref=dbuf.at[crows(c)],
                    send_sem=fxs.at[c], recv_sem=dxr.at[c],
                    device_id=x_partner, device_id_type=pl.DeviceIdType.MESH,
                )
                fx.start()
                fwd_x.append(fx)
            out_ref[qrows(q_zn, c), :] = (
                x_ref[0, qrows(q_zn, c), my_cols] + zbuf[crows(c), :]
            )

        fwd_z = []
        for c in range(C):
            swap_x[c].wait_recv()
            if c >= H:
                fz = pltpu.make_async_remote_copy(
                    src_ref=xbuf.at[crows(c)], dst_ref=dbuf.at[crows(c)],
                    send_sem=fzs.at[c - H], recv_sem=dzr.at[c - H],
                    device_id=z_partner, device_id_type=pl.DeviceIdType.MESH,
                )
                fz.start()
                fwd_z.append(fz)
            out_ref[qrows(q_xn, c), :] = (
                x_ref[0, qrows(q_xn, c), my_cols] + xbuf[crows(c), :]
            )

        for c in range(C):
            recv = pltpu.make_async_remote_copy(
                src_ref=dbuf.at[crows(c)], dst_ref=dbuf.at[crows(c)],
                send_sem=fxs.at[0],
                recv_sem=dxr.at[c] if c < H else dzr.at[c - H],
                device_id=x_partner, device_id_type=pl.DeviceIdType.MESH,
            )
            recv.wait_recv()
            out_ref[qrows(q_dg, c), :] = (
                x_ref[0, qrows(q_dg, c), my_cols] + dbuf[crows(c), :]
            )

        for r in y_rdmas + swap_x + swap_z + fwd_x + fwd_z:
            r.wait_send()

    return pl.pallas_call(
        body,
        out_shape=jax.ShapeDtypeStruct((m, half_n), jnp.float32),
        in_specs=[pl.BlockSpec(memory_space=pltpu.VMEM)],
        out_specs=pl.BlockSpec(memory_space=pltpu.VMEM),
        scratch_shapes=[
            pltpu.VMEM((quarter, half_n), jnp.float32),
            pltpu.VMEM((quarter, half_n), jnp.float32),
            pltpu.VMEM((quarter, half_n), jnp.float32),
            pltpu.VMEM((quarter, half_n), jnp.float32),
            pltpu.SemaphoreType.DMA((C,)),
            pltpu.SemaphoreType.DMA((C,)),
            pltpu.SemaphoreType.DMA((C,)),
            pltpu.SemaphoreType.DMA((C,)),
            pltpu.SemaphoreType.DMA((C,)),
            pltpu.SemaphoreType.DMA((C,)),
            pltpu.SemaphoreType.DMA((H,)),
            pltpu.SemaphoreType.DMA((H,)),
            pltpu.SemaphoreType.DMA((H,)),
            pltpu.SemaphoreType.DMA((H,)),
        ],
        compiler_params=pltpu.CompilerParams(collective_id=0),
    )(x)
